# baseline (speedup 1.0000x reference)
import functools

import jax
import jax.numpy as jnp
from jax import lax
from jax.experimental import pallas as pl
from jax.experimental.pallas import tpu as pltpu
from jax.experimental.pallas import tpu_sc as plsc

NC = 2
NS = 16
NW = NC * NS
CHUNK = 1280
NBUF = 4


def _gather_kernel(table, idx_flat, out_type, emb_dim):
    total = idx_flat.shape[0]
    per_w = total // NW
    steps = per_w // CHUNK
    mesh = plsc.VectorSubcoreMesh(core_axis_name="c", subcore_axis_name="s")

    scratch = (
        [pltpu.VMEM((CHUNK,), jnp.int32) for _ in range(NBUF)]
        + [pltpu.VMEM((CHUNK, emb_dim), jnp.float32) for _ in range(NBUF)]
        + [pltpu.SemaphoreType.DMA] * (3 * NBUF)
    )

    @functools.partial(
        pl.kernel,
        mesh=mesh,
        out_type=out_type,
        compiler_params=pltpu.CompilerParams(use_tc_tiling_on_sc=False),
        scratch_types=scratch,
    )
    def k(table_hbm, idx_hbm, out_hbm, i0, i1, i2, i3, r0, r1, r2, r3,
          si0, si1, si2, si3, sg0, sg1, sg2, sg3, so0, so1, so2, so3):
        idx_v = [i0, i1, i2, i3]
        rows_v = [r0, r1, r2, r3]
        sem_i = [si0, si1, si2, si3]
        sem_g = [sg0, sg1, sg2, sg3]
        sem_o = [so0, so1, so2, so3]
        wid = lax.axis_index("s") * NC + lax.axis_index("c")
        base0 = wid * per_w

        def idx_load(chunk, b):
            pltpu.async_copy(idx_hbm.at[pl.ds(base0 + chunk * CHUNK, CHUNK)],
                             idx_v[b], sem_i[b])

        def idx_wait(b):
            pltpu.make_async_copy(idx_hbm.at[pl.ds(0, CHUNK)], idx_v[b],
                                  sem_i[b]).wait()

        def gather_fire(b):
            pltpu.async_copy(table_hbm.at[idx_v[b]], rows_v[b], sem_g[b])

        def gather_wait(b):
            pltpu.make_async_copy(table_hbm.at[idx_v[b]], rows_v[b],
                                  sem_g[b]).wait()

        def store_fire(chunk, b):
            pltpu.async_copy(rows_v[b],
                             out_hbm.at[pl.ds(base0 + chunk * CHUNK, CHUNK)],
                             sem_o[b])

        def store_wait(b):
            pltpu.make_async_copy(rows_v[b], out_hbm.at[pl.ds(0, CHUNK)],
                                  sem_o[b]).wait()

        idx_load(0, 0)
        idx_load(1, 1)
        idx_load(2, 2)
        idx_wait(0)
        gather_fire(0)
        idx_wait(1)
        gather_fire(1)

        @pl.loop(0, steps)
        def _(g):
            b = lax.rem(g, NBUF)

            def on_buf(bg):
                b2 = (bg + 2) % NBUF
                b3 = (bg + 3) % NBUF

                @pl.when(g + 2 < steps)
                def _():
                    idx_wait(b2)

                    @pl.when(g >= 2)
                    def _():
                        store_wait(b2)

                    gather_fire(b2)

                gather_wait(bg)
                store_fire(g, bg)

                @pl.when(g + 3 < steps)
                def _():
                    idx_load(g + 3, b3)

            for r in range(NBUF):
                @pl.when(b == r)
                def _(r=r):
                    on_buf(r)

        for b in range(NBUF):
            store_wait(b)

    return k(table, idx_flat)


def kernel(indices, table):
    n_rows, n_cols = indices.shape
    emb_dim = table.shape[1]
    total = n_rows * n_cols
    idx_flat = indices.astype(jnp.int32).reshape(total)
    out2d = jax.ShapeDtypeStruct((total, emb_dim), jnp.float32)
    out = _gather_kernel(table, idx_flat, out2d, emb_dim)
    return out.reshape(n_rows, n_cols, emb_dim)


# speedup vs baseline: 1.0007x; 1.0007x over previous
import functools

import jax
import jax.numpy as jnp
from jax import lax
from jax.experimental import pallas as pl
from jax.experimental.pallas import tpu as pltpu
from jax.experimental.pallas import tpu_sc as plsc

NC = 2
NS = 16
NW = NC * NS
CHUNK = 512
NBUF = 8
GLAG = 4


def _gather_kernel(table, idx_flat, out_type, emb_dim):
    total = idx_flat.shape[0]
    per_w = total // NW
    steps = per_w // CHUNK
    assert steps >= NBUF
    mesh = plsc.VectorSubcoreMesh(core_axis_name="c", subcore_axis_name="s")

    scratch = (
        [pltpu.VMEM((CHUNK,), jnp.int32) for _ in range(NBUF)]
        + [pltpu.VMEM((CHUNK, emb_dim), jnp.float32) for _ in range(NBUF)]
        + [pltpu.SemaphoreType.DMA] * (3 * NBUF)
    )

    @functools.partial(
        pl.kernel,
        mesh=mesh,
        out_type=out_type,
        compiler_params=pltpu.CompilerParams(use_tc_tiling_on_sc=False),
        scratch_types=scratch,
    )
    def k(table_hbm, idx_hbm, out_hbm, *scr):
        idx_v = scr[:NBUF]
        rows_v = scr[NBUF:2 * NBUF]
        sem_i = scr[2 * NBUF:3 * NBUF]
        sem_g = scr[3 * NBUF:4 * NBUF]
        sem_o = scr[4 * NBUF:5 * NBUF]
        wid = lax.axis_index("s") * NC + lax.axis_index("c")
        base0 = wid * per_w

        def idx_load(chunk, b):
            pltpu.async_copy(idx_hbm.at[pl.ds(base0 + chunk * CHUNK, CHUNK)],
                             idx_v[b], sem_i[b])

        def idx_wait(b):
            pltpu.make_async_copy(idx_hbm.at[pl.ds(0, CHUNK)], idx_v[b],
                                  sem_i[b]).wait()

        def gather_fire(b):
            pltpu.async_copy(table_hbm.at[idx_v[b]], rows_v[b], sem_g[b])

        def gather_wait(b):
            pltpu.make_async_copy(table_hbm.at[idx_v[b]], rows_v[b],
                                  sem_g[b]).wait()

        def store_fire(chunk, b):
            pltpu.async_copy(rows_v[b],
                             out_hbm.at[pl.ds(base0 + chunk * CHUNK, CHUNK)],
                             sem_o[b])

        def store_wait(b):
            pltpu.make_async_copy(rows_v[b], out_hbm.at[pl.ds(0, CHUNK)],
                                  sem_o[b]).wait()

        for c in range(NBUF):
            idx_load(c, c)

        @pl.loop(0, steps)
        def _(g):
            b = lax.rem(g, NBUF)

            def on_buf(bg):
                bl = (bg - GLAG) % NBUF

                @pl.when(g >= NBUF)
                def _():
                    store_wait(bg)

                idx_wait(bg)
                gather_fire(bg)

                @pl.when(g >= GLAG)
                def _():
                    gather_wait(bl)
                    store_fire(g - GLAG, bl)

                    @pl.when(g + (NBUF - GLAG) < steps)
                    def _():
                        idx_load(g + (NBUF - GLAG), bl)

            for r in range(NBUF):
                @pl.when(b == r)
                def _(r=r):
                    on_buf(r)

        for j in range(steps - GLAG, steps):
            bj = j % NBUF
            gather_wait(bj)
            store_fire(j, bj)
        for b in range(NBUF):
            store_wait(b)

    return k(table, idx_flat)


def kernel(indices, table):
    n_rows, n_cols = indices.shape
    emb_dim = table.shape[1]
    total = n_rows * n_cols
    idx_flat = indices.astype(jnp.int32).reshape(total)
    out2d = jax.ShapeDtypeStruct((total, emb_dim), jnp.float32)
    out = _gather_kernel(table, idx_flat, out2d, emb_dim)
    return out.reshape(n_rows, n_cols, emb_dim)
